# baseline (speedup 1.0000x reference)
import functools

import jax
import jax.numpy as jnp
from jax import lax
from jax.experimental import pallas as pl
from jax.experimental.pallas import tpu as pltpu
from jax.experimental.pallas import tpu_sc as plsc

N = 10000
E = 320000
DIM = 128
G = 16
C = 16

NC = 2
NS = 16
NW = NC * NS
EPT = E // NW
CH = 128
NCH = 2 * (-(-EPT // (2 * CH)))
EPT_PAD = NCH * CH
NPAD = 112
NROW = N + NPAD
ZR = NROW // NS
OA = 624

_mesh = plsc.VectorSubcoreMesh(core_axis_name="c", subcore_axis_name="s")


@functools.partial(
    pl.kernel,
    mesh=_mesh,
    out_type=jax.ShapeDtypeStruct((NC, N, DIM), jnp.float32),
    scratch_types=[
        pltpu.VMEM((NCH, CH), jnp.int32),
        pltpu.VMEM((NCH, CH), jnp.int32),
        pltpu.VMEM((CH, DIM), jnp.float32),
        pltpu.VMEM_SHARED((NROW, DIM), jnp.float32),
        pltpu.SemaphoreType.DMA,
    ],
)
def _segsum_sc(h_hbm, src_hbm, dst_hbm, zero_hbm, out_hbm,
               sidx, didx, rows, agg, sem):
    c = lax.axis_index("c")
    s = lax.axis_index("s")
    wid = c * NS + s
    zoff = pl.multiple_of(s * ZR, 8)
    pltpu.sync_copy(zero_hbm.at[pl.ds(zoff, ZR)], agg.at[pl.ds(zoff, ZR)])
    plsc.subcore_barrier()
    pltpu.sync_copy(src_hbm.at[wid], sidx)
    pltpu.sync_copy(dst_hbm.at[wid], didx)

    def body(j, _):
        pltpu.async_copy(h_hbm.at[sidx.at[j]], rows, sem).wait()
        pltpu.sync_copy(rows, agg.at[didx.at[j]], add=True)
        return ()

    lax.fori_loop(0, NCH, body, (), unroll=False)
    plsc.subcore_barrier()
    ooff = pl.multiple_of(s * OA, 8)

    @pl.when(s < NS - 1)
    def _():
        pltpu.sync_copy(agg.at[pl.ds(ooff, OA)],
                        out_hbm.at[c, pl.ds(ooff, OA)])

    @pl.when(s == NS - 1)
    def _():
        pltpu.sync_copy(agg.at[pl.ds((NS - 1) * OA, N - (NS - 1) * OA)],
                        out_hbm.at[c, pl.ds((NS - 1) * OA, N - (NS - 1) * OA)])


def _dense_body(h_ref, p_ref, wa_ref, ba_ref, wb_ref, bb_ref, g_ref, be_ref,
                o_ref):
    z = h_ref[...] + p_ref[0] + p_ref[1]
    z = lax.dot_general(z, wa_ref[...], (((1,), (0,)), ((), ())),
                        preferred_element_type=jnp.float32,
                        precision=lax.Precision.HIGHEST)
    z = jnp.maximum(z + ba_ref[...], 0.0)
    z = lax.dot_general(z, wb_ref[...], (((1,), (0,)), ((), ())),
                        preferred_element_type=jnp.float32,
                        precision=lax.Precision.HIGHEST)
    z = jnp.maximum(z + bb_ref[...], 0.0)
    mu = jnp.mean(z, axis=0, keepdims=True)
    d = z - mu
    var = jnp.mean(d * d, axis=0, keepdims=True)
    o_ref[...] = g_ref[...] * d * lax.rsqrt(var + 1e-5) + be_ref[...]


_dense = pl.pallas_call(
    _dense_body,
    out_shape=jax.ShapeDtypeStruct((N, DIM), jnp.float32),
)


def _head_body(h_ref, b_ref, w1_ref, b1_ref, w2_ref, b2_ref, o_ref):
    oh = (b_ref[...] == lax.broadcasted_iota(jnp.int32, (1, G), 1)
          ).astype(jnp.float32)
    sums = lax.dot_general(oh, h_ref[...], (((0,), (0,)), ((), ())),
                           preferred_element_type=jnp.float32,
                           precision=lax.Precision.HIGHEST)
    ones = jnp.full((N, 1), 1.0, jnp.float32)
    cnt = lax.dot_general(oh, ones, (((0,), (0,)), ((), ())),
                          preferred_element_type=jnp.float32,
                          precision=lax.Precision.HIGHEST)
    p = sums / jnp.maximum(cnt, 1.0)
    p = lax.dot_general(p, w1_ref[...], (((1,), (0,)), ((), ())),
                        preferred_element_type=jnp.float32,
                        precision=lax.Precision.HIGHEST)
    p = jnp.maximum(p + b1_ref[...], 0.0)
    p = lax.dot_general(p, w2_ref[...], (((1,), (0,)), ((), ())),
                        preferred_element_type=jnp.float32,
                        precision=lax.Precision.HIGHEST)
    p = p + b2_ref[...]
    m = jnp.max(p, axis=1, keepdims=True)
    lse = m + jnp.log(jnp.sum(jnp.exp(p - m), axis=1, keepdims=True))
    o_ref[...] = p - lse


_head = pl.pallas_call(
    _head_body,
    out_shape=jax.ShapeDtypeStruct((G, C), jnp.float32),
)


def kernel(x, edge_index, batch,
           W1a, b1a, W1b, b1b, g1, be1,
           W2a, b2a, W2b, b2b, g2, be2,
           W3a, b3a, W3b, b3b, g3, be3,
           W4a, b4a, W4b, b4b, g4, be4,
           W5a, b5a, W5b, b5b, g5, be5,
           fc1_W, fc1_b, fc2_W, fc2_b):
    layers = [(W1a, b1a, W1b, b1b, g1, be1),
              (W2a, b2a, W2b, b2b, g2, be2),
              (W3a, b3a, W3b, b3b, g3, be3),
              (W4a, b4a, W4b, b4b, g4, be4),
              (W5a, b5a, W5b, b5b, g5, be5)]
    src = edge_index[0].reshape(NW, EPT)
    dst = edge_index[1].reshape(NW, EPT)
    pad = EPT_PAD - EPT
    srcp = jnp.concatenate(
        [src, jnp.zeros((NW, pad), jnp.int32)], axis=1).reshape(NW, NCH, CH)
    pad_dst = jnp.broadcast_to(N + (jnp.arange(pad, dtype=jnp.int32) % NPAD),
                               (NW, pad))
    dstp = jnp.concatenate([dst, pad_dst], axis=1).reshape(NW, NCH, CH)
    zero = jnp.zeros((NROW, DIM), jnp.float32)
    h = x
    for (wa, ba, wb, bb, gg, be) in layers:
        p = _segsum_sc(h, srcp, dstp, zero)
        h = _dense(h, p, wa, ba.reshape(1, DIM), wb, bb.reshape(1, DIM),
                   gg.reshape(1, DIM), be.reshape(1, DIM))
    return _head(h, batch.reshape(N, 1), fc1_W, fc1_b.reshape(1, DIM),
                 fc2_W, fc2_b.reshape(1, C))


# speedup vs baseline: 2.3200x; 2.3200x over previous
import functools

import jax
import jax.numpy as jnp
from jax import lax
from jax.experimental import pallas as pl
from jax.experimental.pallas import tpu as pltpu
from jax.experimental.pallas import tpu_sc as plsc

N = 10000
E = 320000
DIM = 128
G = 16
C = 16

NC = 2
NS = 16
NW = NC * NS
EPT = E // NW
CH = 128
NCH = 2 * (-(-EPT // (2 * CH)))
EPT_PAD = NCH * CH
NPAD = 112
NROW = N + NPAD
ZR = NROW // NS
OA = 624

_mesh = plsc.VectorSubcoreMesh(core_axis_name="c", subcore_axis_name="s")


@functools.partial(
    pl.kernel,
    mesh=_mesh,
    out_type=jax.ShapeDtypeStruct((NC, N, DIM), jnp.float32),
    scratch_types=[
        pltpu.VMEM((NCH, CH), jnp.int32),
        pltpu.VMEM((NCH, CH), jnp.int32),
        pltpu.VMEM((CH, DIM), jnp.float32),
        pltpu.VMEM_SHARED((NROW, DIM), jnp.float32),
        pltpu.SemaphoreType.DMA,
    ],
)
def _segsum_sc(h_hbm, src_hbm, dst_hbm, zero_hbm, out_hbm,
               sidx, didx, rows, agg, sem):
    c = lax.axis_index("c")
    s = lax.axis_index("s")
    wid = c * NS + s
    zoff = pl.multiple_of(s * ZR, 8)
    pltpu.sync_copy(zero_hbm.at[pl.ds(zoff, ZR)], agg.at[pl.ds(zoff, ZR)])
    plsc.subcore_barrier()
    pltpu.sync_copy(src_hbm.at[wid], sidx)
    pltpu.sync_copy(dst_hbm.at[wid], didx)

    def body(j, _):
        pltpu.async_copy(h_hbm.at[sidx.at[j]], rows, sem).wait()
        pltpu.sync_copy(rows, agg.at[didx.at[j]], add=True)
        return ()

    lax.fori_loop(0, NCH, body, (), unroll=False)
    plsc.subcore_barrier()
    ooff = pl.multiple_of(s * OA, 8)

    @pl.when(s < NS - 1)
    def _():
        pltpu.sync_copy(agg.at[pl.ds(ooff, OA)],
                        out_hbm.at[c, pl.ds(ooff, OA)])

    @pl.when(s == NS - 1)
    def _():
        pltpu.sync_copy(agg.at[pl.ds((NS - 1) * OA, N - (NS - 1) * OA)],
                        out_hbm.at[c, pl.ds((NS - 1) * OA, N - (NS - 1) * OA)])


def _dense_body(h_ref, p_ref, wa_ref, ba_ref, wb_ref, bb_ref, g_ref, be_ref,
                o_ref):
    z = h_ref[...] + p_ref[0] + p_ref[1]
    z = lax.dot_general(z, wa_ref[...], (((1,), (0,)), ((), ())),
                        preferred_element_type=jnp.float32,
                        precision=lax.Precision.HIGHEST)
    z = jnp.maximum(z + ba_ref[...], 0.0)
    z = lax.dot_general(z, wb_ref[...], (((1,), (0,)), ((), ())),
                        preferred_element_type=jnp.float32,
                        precision=lax.Precision.HIGHEST)
    z = jnp.maximum(z + bb_ref[...], 0.0)
    mu = jnp.mean(z, axis=0, keepdims=True)
    d = z - mu
    var = jnp.mean(d * d, axis=0, keepdims=True)
    o_ref[...] = g_ref[...] * d * lax.rsqrt(var + 1e-5) + be_ref[...]


_dense = pl.pallas_call(
    _dense_body,
    out_shape=jax.ShapeDtypeStruct((N, DIM), jnp.float32),
)


def _head_body(h_ref, b_ref, w1_ref, b1_ref, w2_ref, b2_ref, o_ref):
    oh = (b_ref[...] == lax.broadcasted_iota(jnp.int32, (1, G), 1)
          ).astype(jnp.float32)
    sums = lax.dot_general(oh, h_ref[...], (((0,), (0,)), ((), ())),
                           preferred_element_type=jnp.float32,
                           precision=lax.Precision.HIGHEST)
    ones = jnp.full((N, 1), 1.0, jnp.float32)
    cnt = lax.dot_general(oh, ones, (((0,), (0,)), ((), ())),
                          preferred_element_type=jnp.float32,
                          precision=lax.Precision.HIGHEST)
    p = sums / jnp.maximum(cnt, 1.0)
    p = lax.dot_general(p, w1_ref[...], (((1,), (0,)), ((), ())),
                        preferred_element_type=jnp.float32,
                        precision=lax.Precision.HIGHEST)
    p = jnp.maximum(p + b1_ref[...], 0.0)
    p = lax.dot_general(p, w2_ref[...], (((1,), (0,)), ((), ())),
                        preferred_element_type=jnp.float32,
                        precision=lax.Precision.HIGHEST)
    p = p + b2_ref[...]
    m = jnp.max(p, axis=1, keepdims=True)
    lse = m + jnp.log(jnp.sum(jnp.exp(p - m), axis=1, keepdims=True))
    o_ref[...] = p - lse


_head = pl.pallas_call(
    _head_body,
    out_shape=jax.ShapeDtypeStruct((G, C), jnp.float32),
)


def kernel(x, edge_index, batch,
           W1a, b1a, W1b, b1b, g1, be1,
           W2a, b2a, W2b, b2b, g2, be2,
           W3a, b3a, W3b, b3b, g3, be3,
           W4a, b4a, W4b, b4b, g4, be4,
           W5a, b5a, W5b, b5b, g5, be5,
           fc1_W, fc1_b, fc2_W, fc2_b):
    layers = [(W1a, b1a, W1b, b1b, g1, be1),
              (W2a, b2a, W2b, b2b, g2, be2),
              (W3a, b3a, W3b, b3b, g3, be3),
              (W4a, b4a, W4b, b4b, g4, be4),
              (W5a, b5a, W5b, b5b, g5, be5)]
    src = edge_index[0].reshape(NW, EPT)
    dst = edge_index[1].reshape(NW, EPT)
    pad = EPT_PAD - EPT
    pad_src = (jnp.arange(NW, dtype=jnp.int32)[:, None] * 997
               + jnp.arange(pad, dtype=jnp.int32)[None, :] * 31) % N
    srcp = jnp.concatenate([src, pad_src], axis=1).reshape(NW, NCH, CH)
    pad_dst = jnp.broadcast_to(N + (jnp.arange(pad, dtype=jnp.int32) % NPAD),
                               (NW, pad))
    dstp = jnp.concatenate([dst, pad_dst], axis=1).reshape(NW, NCH, CH)
    zero = jnp.zeros((NROW, DIM), jnp.float32)
    h = x
    for (wa, ba, wb, bb, gg, be) in layers:
        p = _segsum_sc(h, srcp, dstp, zero)
        h = _dense(h, p, wa, ba.reshape(1, DIM), wb, bb.reshape(1, DIM),
                   gg.reshape(1, DIM), be.reshape(1, DIM))
    return _head(h, batch.reshape(N, 1), fc1_W, fc1_b.reshape(1, DIM),
                 fc2_W, fc2_b.reshape(1, C))


# speedup vs baseline: 3.3850x; 1.4591x over previous
import functools

import jax
import jax.numpy as jnp
from jax import lax
from jax.experimental import pallas as pl
from jax.experimental.pallas import tpu as pltpu
from jax.experimental.pallas import tpu_sc as plsc

N = 10000
E = 320000
DIM = 128
G = 16
C = 16

NC = 2
NS = 16
NW = NC * NS
EPT = E // NW
CH = 128
NCH = 2 * (-(-EPT // (2 * CH)))
EPT_PAD = NCH * CH
NPAD = 112
NROW = N + NPAD
ZR = NROW // NS
OA = 624

_mesh = plsc.VectorSubcoreMesh(core_axis_name="c", subcore_axis_name="s")


@functools.partial(
    pl.kernel,
    mesh=_mesh,
    out_type=jax.ShapeDtypeStruct((NC, N, DIM), jnp.float32),
    scratch_types=[
        pltpu.VMEM((NCH // 2, CH), jnp.int32),
        pltpu.VMEM((NCH // 2, CH), jnp.int32),
        pltpu.VMEM((CH, DIM), jnp.float32),
        pltpu.VMEM((CH, DIM), jnp.float32),
        pltpu.VMEM_SHARED((NROW, DIM), jnp.float32),
        pltpu.SemaphoreType.DMA,
        pltpu.SemaphoreType.DMA,
    ],
)
def _segsum_sc(h_hbm, src_hbm, dst_hbm, zero_hbm, out_hbm,
               sidx, didx, rows0, rows1, agg, sg0, sg1):
    c = lax.axis_index("c")
    s = lax.axis_index("s")
    wid = c * NS + s
    zoff = pl.multiple_of(s * ZR, 8)
    pltpu.sync_copy(zero_hbm.at[pl.ds(zoff, ZR)], agg.at[pl.ds(zoff, ZR)])
    plsc.subcore_barrier()
    HNCH = NCH // 2

    def _pass(base):
        pltpu.sync_copy(src_hbm.at[wid, pl.ds(base, HNCH)], sidx)
        pltpu.sync_copy(dst_hbm.at[wid, pl.ds(base, HNCH)], didx)
        pltpu.async_copy(h_hbm.at[sidx.at[0]], rows0, sg0)
        pltpu.async_copy(h_hbm.at[sidx.at[1]], rows1, sg1)

        def body(j2, _):
            j = j2 * 2
            pltpu.make_async_copy(h_hbm.at[sidx.at[j]], rows0, sg0).wait()
            pltpu.sync_copy(rows0, agg.at[didx.at[j]], add=True)

            @pl.when(j + 2 < HNCH)
            def _():
                pltpu.async_copy(h_hbm.at[sidx.at[j + 2]], rows0, sg0)

            pltpu.make_async_copy(h_hbm.at[sidx.at[j + 1]], rows1, sg1).wait()
            pltpu.sync_copy(rows1, agg.at[didx.at[j + 1]], add=True)

            @pl.when(j + 3 < HNCH)
            def _():
                pltpu.async_copy(h_hbm.at[sidx.at[j + 3]], rows1, sg1)

            return ()

        lax.fori_loop(0, HNCH // 2, body, (), unroll=False)

    _pass(0)
    _pass(HNCH)
    plsc.subcore_barrier()
    ooff = pl.multiple_of(s * OA, 8)

    @pl.when(s < NS - 1)
    def _():
        pltpu.sync_copy(agg.at[pl.ds(ooff, OA)],
                        out_hbm.at[c, pl.ds(ooff, OA)])

    @pl.when(s == NS - 1)
    def _():
        pltpu.sync_copy(agg.at[pl.ds((NS - 1) * OA, N - (NS - 1) * OA)],
                        out_hbm.at[c, pl.ds((NS - 1) * OA, N - (NS - 1) * OA)])


def _dense_body(h_ref, p_ref, wa_ref, ba_ref, wb_ref, bb_ref, g_ref, be_ref,
                o_ref):
    z = h_ref[...] + p_ref[0] + p_ref[1]
    z = lax.dot_general(z, wa_ref[...], (((1,), (0,)), ((), ())),
                        preferred_element_type=jnp.float32,
                        precision=lax.Precision.HIGHEST)
    z = jnp.maximum(z + ba_ref[...], 0.0)
    z = lax.dot_general(z, wb_ref[...], (((1,), (0,)), ((), ())),
                        preferred_element_type=jnp.float32,
                        precision=lax.Precision.HIGHEST)
    z = jnp.maximum(z + bb_ref[...], 0.0)
    mu = jnp.mean(z, axis=0, keepdims=True)
    d = z - mu
    var = jnp.mean(d * d, axis=0, keepdims=True)
    o_ref[...] = g_ref[...] * d * lax.rsqrt(var + 1e-5) + be_ref[...]


_dense = pl.pallas_call(
    _dense_body,
    out_shape=jax.ShapeDtypeStruct((N, DIM), jnp.float32),
)


def _head_body(h_ref, b_ref, w1_ref, b1_ref, w2_ref, b2_ref, o_ref):
    oh = (b_ref[...] == lax.broadcasted_iota(jnp.int32, (1, G), 1)
          ).astype(jnp.float32)
    sums = lax.dot_general(oh, h_ref[...], (((0,), (0,)), ((), ())),
                           preferred_element_type=jnp.float32,
                           precision=lax.Precision.HIGHEST)
    ones = jnp.full((N, 1), 1.0, jnp.float32)
    cnt = lax.dot_general(oh, ones, (((0,), (0,)), ((), ())),
                          preferred_element_type=jnp.float32,
                          precision=lax.Precision.HIGHEST)
    p = sums / jnp.maximum(cnt, 1.0)
    p = lax.dot_general(p, w1_ref[...], (((1,), (0,)), ((), ())),
                        preferred_element_type=jnp.float32,
                        precision=lax.Precision.HIGHEST)
    p = jnp.maximum(p + b1_ref[...], 0.0)
    p = lax.dot_general(p, w2_ref[...], (((1,), (0,)), ((), ())),
                        preferred_element_type=jnp.float32,
                        precision=lax.Precision.HIGHEST)
    p = p + b2_ref[...]
    m = jnp.max(p, axis=1, keepdims=True)
    lse = m + jnp.log(jnp.sum(jnp.exp(p - m), axis=1, keepdims=True))
    o_ref[...] = p - lse


_head = pl.pallas_call(
    _head_body,
    out_shape=jax.ShapeDtypeStruct((G, C), jnp.float32),
)


def kernel(x, edge_index, batch,
           W1a, b1a, W1b, b1b, g1, be1,
           W2a, b2a, W2b, b2b, g2, be2,
           W3a, b3a, W3b, b3b, g3, be3,
           W4a, b4a, W4b, b4b, g4, be4,
           W5a, b5a, W5b, b5b, g5, be5,
           fc1_W, fc1_b, fc2_W, fc2_b):
    layers = [(W1a, b1a, W1b, b1b, g1, be1),
              (W2a, b2a, W2b, b2b, g2, be2),
              (W3a, b3a, W3b, b3b, g3, be3),
              (W4a, b4a, W4b, b4b, g4, be4),
              (W5a, b5a, W5b, b5b, g5, be5)]
    src = edge_index[0].reshape(NW, EPT)
    dst = edge_index[1].reshape(NW, EPT)
    pad = EPT_PAD - EPT
    pad_src = (jnp.arange(NW, dtype=jnp.int32)[:, None] * 997
               + jnp.arange(pad, dtype=jnp.int32)[None, :] * 31) % N
    srcp = jnp.concatenate([src, pad_src], axis=1).reshape(NW, NCH, CH)
    pad_dst = jnp.broadcast_to(N + (jnp.arange(pad, dtype=jnp.int32) % NPAD),
                               (NW, pad))
    dstp = jnp.concatenate([dst, pad_dst], axis=1).reshape(NW, NCH, CH)
    zero = jnp.zeros((NROW, DIM), jnp.float32)
    h = x
    for (wa, ba, wb, bb, gg, be) in layers:
        p = _segsum_sc(h, srcp, dstp, zero)
        h = _dense(h, p, wa, ba.reshape(1, DIM), wb, bb.reshape(1, DIM),
                   gg.reshape(1, DIM), be.reshape(1, DIM))
    return _head(h, batch.reshape(N, 1), fc1_W, fc1_b.reshape(1, DIM),
                 fc2_W, fc2_b.reshape(1, C))
